# baseline (speedup 1.0000x reference)
import jax
import jax.numpy as jnp
from jax import lax
from jax.experimental import pallas as pl
from jax.experimental.pallas import tpu as pltpu
from jax.experimental.pallas import tpu_sc as plsc

N_NODES = 10000
D = 128
NUM_REL = 6
COL = 16
NPASS = D // COL
PASS_PC = NPASS // 2
ROWS = NUM_REL * N_NODES
PAD_ROWS = 512
ACC_ROWS = ROWS + PAD_ROWS
NTILES = 16
NODES_PT = N_NODES // NTILES
NODES_PT_PAD = 640
ZERO_PT = ACC_ROWS // NTILES

E_RAW = 320000
E_TOT = E_RAW + N_NODES
SUB = 128
SUBS_PER_CHUNK = 8
CHUNK = SUB * SUBS_PER_CHUNK
CHUNKS_PT = 21
TRIPLES = CHUNKS_PT // 3
EDGES_PT = CHUNK * CHUNKS_PT
E_PAD = EDGES_PT * NTILES
IDX_ROWS_PT = EDGES_PT // SUB
IDX_ROWS_TOT = E_PAD // SUB

ZBUF_ROWS = 256
ZCOPIES = ZERO_PT // ZBUF_ROWS
ZTAIL = ZERO_PT - ZCOPIES * ZBUF_ROWS


def _sc_body(feat_all, comb1, sidx, comb2,
             h2, h1_all,
             acc, zbuf, obuf, rows, gis, sis, recip_v, sem, sI, sG, sS):
  c = lax.axis_index("c")
  s = lax.axis_index("s")
  irow0 = s * IDX_ROWS_PT
  nbase = s * NODES_PT
  zbase = s * ZERO_PT

  zvec = jnp.zeros((16,), jnp.float32)
  lane = lax.iota(jnp.int32, 16)
  onevec = jnp.where(lane == 0, 1.0, 0.0).astype(jnp.float32)

  def zero_acc():
    descs = [pltpu.async_copy(
        zbuf, acc.at[pl.ds(zbase + m * ZBUF_ROWS, ZBUF_ROWS)], sem)
        for m in range(ZCOPIES)]
    descs.append(pltpu.async_copy(
        zbuf.at[pl.ds(0, ZTAIL)],
        acc.at[pl.ds(zbase + ZCOPIES * ZBUF_ROWS, ZTAIL)], sem))
    for d in descs:
      d.wait()

  def scatter_pass(table, gidx, goff):
    def apply_off(gi):
      for j in range(SUBS_PER_CHUNK):
        for q in range(SUB // 16):
          sl = pl.ds(q * 16, 16)
          gi[2 * j, sl] = gi[2 * j, sl] + goff

    def fire_idx(k, t):
      r0 = irow0 + k * SUBS_PER_CHUNK
      return pltpu.async_copy(
          gidx.at[pl.ds(2 * r0, 2 * SUBS_PER_CHUNK)], gis[t], sI[t])

    def fire_gather(t):
      for j in range(SUBS_PER_CHUNK):
        pltpu.async_copy(table.at[gis[t].at[2 * j]],
                         rows[t].at[pl.ds(j * SUB, SUB)], sG[t])

    def fire_scatter(t):
      for j in range(SUBS_PER_CHUNK):
        pltpu.async_copy(rows[t].at[pl.ds(j * SUB, SUB)],
                         acc.at[gis[t].at[2 * j + 1]], sS[t], add=True)

    def drain_rows(s_x):
      pltpu.make_async_copy(table.at[pl.ds(0, CHUNK)], rows[0], s_x).wait()

    def stage_idx_gather(k, t):
      fire_idx(k, t).wait()
      apply_off(gis[t])
      fire_gather(t)

    stage_idx_gather(0, 0)

    def body(m, _):
      for t in range(3):
        nxt = (t + 1) % 3
        if t < 2:
          @pl.when(m > 0)
          def _():
            drain_rows(sS[nxt])
          stage_idx_gather(3 * m + t + 1, nxt)
        else:
          drain_rows(sS[nxt])
          @pl.when(m < TRIPLES - 1)
          def _():
            stage_idx_gather(3 * m + t + 1, nxt)
        drain_rows(sG[t])
        fire_scatter(t)
      return 0
    lax.fori_loop(0, TRIPLES, body, 0)
    drain_rows(sS[(CHUNKS_PT - 2) % 3])
    drain_rows(sS[(CHUNKS_PT - 1) % 3])

  def count_pass():
    def fire_i(k, t):
      r0 = irow0 + k * SUBS_PER_CHUNK
      return pltpu.async_copy(sidx.at[pl.ds(r0, SUBS_PER_CHUNK)],
                              sis[t], sI[t])

    def drain_i(t):
      pltpu.make_async_copy(sidx.at[pl.ds(irow0, SUBS_PER_CHUNK)],
                            sis[0], sI[t]).wait()

    def fire_s(t):
      for j in range(SUBS_PER_CHUNK):
        pltpu.async_copy(obuf, acc.at[sis[t].at[j]], sS[t], add=True)

    def drain_s(t):
      pltpu.make_async_copy(feat_all.at[pl.ds(0, CHUNK)], rows[0],
                            sS[t]).wait()

    fire_i(0, 0).wait()
    def body(m, _):
      for t in range(3):
        k = 3 * m + t
        nxt = (t + 1) % 3
        if t < 2:
          @pl.when(m > 0)
          def _():
            drain_s(nxt)
          fire_i(k + 1, nxt)
        else:
          drain_s(nxt)
          @pl.when(m < TRIPLES - 1)
          def _():
            fire_i(k + 1, nxt)
        @pl.when(k > 0)
        def _():
          drain_i(t)
        fire_s(t)
      return 0
    lax.fori_loop(0, TRIPLES, body, 0)
    drain_s((CHUNKS_PT - 2) % 3)
    drain_s((CHUNKS_PT - 1) % 3)

  def make_divide(buf):
    def divide(r):
      def divgrp(k, _):
        rvec = recip_v[pl.ds(r * NODES_PT_PAD + k * 16, 16)]
        base = k * 16
        for j in range(16):
          rv = rvec[j]
          buf[base + j, 0:16] = buf[base + j, 0:16] * rv
        return 0
      lax.fori_loop(0, NODES_PT_PAD // 16, divgrp, 0)
    return divide

  divides = [make_divide(rows[0]), make_divide(rows[1]),
             make_divide(rows[2])]

  def fire_zero_rel(r):
    row0 = r * N_NODES + nbase
    ds = [pltpu.async_copy(zbuf, acc.at[pl.ds(row0, ZBUF_ROWS)], sS[0]),
          pltpu.async_copy(zbuf,
                           acc.at[pl.ds(row0 + ZBUF_ROWS, ZBUF_ROWS)], sS[0]),
          pltpu.async_copy(zbuf.at[pl.ds(0, NODES_PT - 2 * ZBUF_ROWS)],
                           acc.at[pl.ds(row0 + 2 * ZBUF_ROWS,
                                        NODES_PT - 2 * ZBUF_ROWS)], sS[0])]
    return ds

  def fire_zero_dump():
    return [pltpu.async_copy(
        zbuf.at[pl.ds(0, PAD_ROWS // NTILES)],
        acc.at[pl.ds(ROWS + s * (PAD_ROWS // NTILES), PAD_ROWS // NTILES)],
        sS[0])]

  def fire_in(r):
    row0 = r * N_NODES + nbase
    return pltpu.async_copy(acc.at[pl.ds(row0, NODES_PT)],
                            rows[r % 3].at[pl.ds(0, NODES_PT)], sI[r % 3])

  def writeback(dst_for):
    zs = fire_zero_dump()
    ins = {0: fire_in(0)}
    outs = {}
    for r in range(NUM_REL):
      buf = rows[r % 3]
      if r + 1 < NUM_REL:
        if r - 2 >= 0:
          outs[r - 2].wait()
        ins[r + 1] = fire_in(r + 1)
      ins[r].wait()
      zs += fire_zero_rel(r)
      divides[r % 3](r)
      outs[r] = pltpu.async_copy(buf.at[pl.ds(0, NODES_PT)],
                                 dst_for(r), sG[r % 3])
    for r in (NUM_REL - 3, NUM_REL - 2, NUM_REL - 1):
      outs[r].wait()
    for d in zs:
      d.wait()

  def zfill(i, _):
    zbuf[i, 0:16] = zvec
    return 0
  lax.fori_loop(0, ZBUF_ROWS, zfill, 0)
  def ones_fill(i, _):
    obuf[i, 0:16] = onevec
    return 0
  lax.fori_loop(0, SUB, ones_fill, 0)
  zero_acc()
  plsc.subcore_barrier()

  count_pass()
  plsc.subcore_barrier()
  zcol = jnp.zeros((16,), jnp.int32)
  zs0 = fire_zero_dump()
  for r in range(NUM_REL):
    row0 = r * N_NODES + nbase
    pltpu.sync_copy(acc.at[pl.ds(row0, NODES_PT)],
                    rows[0].at[pl.ds(0, NODES_PT)])
    zs0 += fire_zero_rel(r)
    def recipgrp(k, _, r=r):
      ridx = k * 16 + lane
      cv = plsc.load_gather(rows[0], [ridx, zcol])
      rvec = 1.0 / jnp.maximum(cv, 1.0)
      recip_v[pl.ds(r * NODES_PT_PAD + k * 16, 16)] = rvec
      return 0
    lax.fori_loop(0, NODES_PT_PAD // 16, recipgrp, 0)
  for d in zs0:
    d.wait()
  plsc.subcore_barrier()

  def step1(j, _):
    p = c * PASS_PC + j
    scatter_pass(feat_all, comb1, p * N_NODES)
    plsc.subcore_barrier()
    writeback(lambda r: h1_all.at[
        pl.ds(p * ROWS + r * N_NODES + nbase, NODES_PT)])
    plsc.subcore_barrier()
    return 0
  lax.fori_loop(0, PASS_PC, step1, 0)

  def step2(j, _):
    p = c * PASS_PC + j
    scatter_pass(h1_all, comb2, p * ROWS)
    plsc.subcore_barrier()
    writeback(lambda r: h2.at[pl.ds(nbase, NODES_PT),
                              pl.ds(r * D + p * COL, COL)])
    plsc.subcore_barrier()
    return 0
  lax.fori_loop(0, PASS_PC, step2, 0)


def _matmul_body(h2_ref, w_ref, o_ref):
  o_ref[...] = jax.nn.relu(
      jnp.dot(h2_ref[...], w_ref[...], preferred_element_type=jnp.float32))


def kernel(features, edge_index, edge_type, W):
  n = features.shape[0]
  loop = jnp.arange(n, dtype=edge_index.dtype)
  src = jnp.concatenate([edge_index[0], loop])
  dst = jnp.concatenate([edge_index[1], loop])
  etype = jnp.concatenate([edge_type, jnp.zeros((n,), edge_type.dtype)])

  sidx = etype * n + dst
  g2 = etype * n + src

  pad = E_PAD - E_TOT
  padi = jnp.arange(pad, dtype=jnp.int32)
  src_p = jnp.concatenate([src, padi % n]).reshape(E_PAD // SUB, SUB)
  sidx_p = jnp.concatenate([sidx, ROWS + padi % PAD_ROWS]).reshape(
      E_PAD // SUB, SUB)
  g2_p = jnp.concatenate([g2, padi % ROWS]).reshape(E_PAD // SUB, SUB)
  comb1 = jnp.stack([src_p, sidx_p], axis=1).reshape(2 * E_PAD // SUB, SUB)
  comb2 = jnp.stack([g2_p, sidx_p], axis=1).reshape(2 * E_PAD // SUB, SUB)

  feat_all = jnp.transpose(
      features.reshape(n, NPASS, COL), (1, 0, 2)).reshape(NPASS * n, COL)

  mesh = plsc.VectorSubcoreMesh(core_axis_name="c", subcore_axis_name="s")
  f32 = jnp.float32
  sc = pl.kernel(
      _sc_body,
      out_type=(
          jax.ShapeDtypeStruct((n, NUM_REL * D), f32),
          jax.ShapeDtypeStruct((NPASS * ROWS, COL), f32),
      ),
      mesh=mesh,
      scratch_types=[
          pltpu.VMEM_SHARED((ACC_ROWS, COL), f32),
          pltpu.VMEM((ZBUF_ROWS, COL), f32),
          pltpu.VMEM((SUB, COL), f32),
          [pltpu.VMEM((CHUNK, COL), f32) for _ in range(3)],
          [pltpu.VMEM((2 * SUBS_PER_CHUNK, SUB), jnp.int32)
           for _ in range(3)],
          [pltpu.VMEM((SUBS_PER_CHUNK, SUB), jnp.int32)
           for _ in range(3)],
          pltpu.VMEM((NUM_REL * NODES_PT_PAD,), f32),
          pltpu.SemaphoreType.DMA,
          [pltpu.SemaphoreType.DMA for _ in range(3)],
          [pltpu.SemaphoreType.DMA for _ in range(3)],
          [pltpu.SemaphoreType.DMA for _ in range(3)],
      ],
      compiler_params=pltpu.CompilerParams(use_tc_tiling_on_sc=False,
                                           needs_layout_passes=False),
      name="rhdc_sc_diffusion",
  )
  h2, _ = sc(feat_all, comb1, sidx_p, comb2)

  w_flat = W.reshape(NUM_REL * D, D)
  block_m = 400
  out = pl.pallas_call(
      _matmul_body,
      grid=(n // block_m,),
      in_specs=[
          pl.BlockSpec((block_m, NUM_REL * D), lambda i: (i, 0)),
          pl.BlockSpec((NUM_REL * D, D), lambda i: (0, 0)),
      ],
      out_specs=pl.BlockSpec((block_m, D), lambda i: (i, 0)),
      out_shape=jax.ShapeDtypeStruct((n, D), jnp.float32),
  )(h2, w_flat)
  return out


# speedup vs baseline: 1.0032x; 1.0032x over previous
import jax
import jax.numpy as jnp
from jax import lax
from jax.experimental import pallas as pl
from jax.experimental.pallas import tpu as pltpu
from jax.experimental.pallas import tpu_sc as plsc

N_NODES = 10000
D = 128
NUM_REL = 6
COL = 16
NPASS = D // COL
PASS_PC = NPASS // 2
ROWS = NUM_REL * N_NODES
PAD_ROWS = 512
ACC_ROWS = ROWS + PAD_ROWS
NTILES = 16
NODES_PT = N_NODES // NTILES
NODES_PT_PAD = 640
ZERO_PT = ACC_ROWS // NTILES

E_RAW = 320000
E_TOT = E_RAW + N_NODES
SUB = 128
SUBS_PER_CHUNK = 8
CHUNK = SUB * SUBS_PER_CHUNK
CHUNKS_PT = 21
TRIPLES = CHUNKS_PT // 3
EDGES_PT = CHUNK * CHUNKS_PT
E_PAD = EDGES_PT * NTILES
IDX_ROWS_PT = EDGES_PT // SUB
IDX_ROWS_TOT = E_PAD // SUB

ZBUF_ROWS = 256
ZCOPIES = ZERO_PT // ZBUF_ROWS
ZTAIL = ZERO_PT - ZCOPIES * ZBUF_ROWS


def _sc_body(feat_all, gidx1, sidx, gidx2,
             h2, h1_all,
             acc, zbuf, obuf, rows, gis, sis, recip_v, sem, sI, sG, sS):
  c = lax.axis_index("c")
  s = lax.axis_index("s")
  irow0 = s * IDX_ROWS_PT
  nbase = s * NODES_PT
  zbase = s * ZERO_PT

  zvec = jnp.zeros((16,), jnp.float32)
  lane = lax.iota(jnp.int32, 16)
  onevec = jnp.where(lane == 0, 1.0, 0.0).astype(jnp.float32)

  def zero_acc():
    descs = [pltpu.async_copy(
        zbuf, acc.at[pl.ds(zbase + m * ZBUF_ROWS, ZBUF_ROWS)], sem)
        for m in range(ZCOPIES)]
    descs.append(pltpu.async_copy(
        zbuf.at[pl.ds(0, ZTAIL)],
        acc.at[pl.ds(zbase + ZCOPIES * ZBUF_ROWS, ZTAIL)], sem))
    for d in descs:
      d.wait()

  def scatter_pass(table, gidx, goff):
    def apply_off(gi):
      for j in range(SUBS_PER_CHUNK):
        for q in range(SUB // 16):
          sl = pl.ds(q * 16, 16)
          gi[j, sl] = gi[j, sl] + goff

    def fire_idx(k, t):
      r0 = irow0 + k * SUBS_PER_CHUNK
      return (pltpu.async_copy(gidx.at[pl.ds(r0, SUBS_PER_CHUNK)],
                               gis[t], sI[t]),
              pltpu.async_copy(sidx.at[pl.ds(r0, SUBS_PER_CHUNK)],
                               sis[t], sI[t]))

    def fire_gather(t):
      for j in range(SUBS_PER_CHUNK):
        pltpu.async_copy(table.at[gis[t].at[j]],
                         rows[t].at[pl.ds(j * SUB, SUB)], sG[t])

    def fire_scatter(t):
      for j in range(SUBS_PER_CHUNK):
        pltpu.async_copy(rows[t].at[pl.ds(j * SUB, SUB)],
                         acc.at[sis[t].at[j]], sS[t], add=True)

    def drain_rows(s_x):
      pltpu.make_async_copy(table.at[pl.ds(0, CHUNK)], rows[0], s_x).wait()

    def stage_idx_gather(k, t):
      d1, d2 = fire_idx(k, t)
      d1.wait()
      d2.wait()
      apply_off(gis[t])
      fire_gather(t)

    stage_idx_gather(0, 0)

    def body(m, _):
      for t in range(3):
        nxt = (t + 1) % 3
        if t < 2:
          @pl.when(m > 0)
          def _():
            drain_rows(sS[nxt])
          stage_idx_gather(3 * m + t + 1, nxt)
        else:
          drain_rows(sS[nxt])
          @pl.when(m < TRIPLES - 1)
          def _():
            stage_idx_gather(3 * m + t + 1, nxt)
        drain_rows(sG[t])
        fire_scatter(t)
      return 0
    lax.fori_loop(0, TRIPLES, body, 0)
    drain_rows(sS[(CHUNKS_PT - 2) % 3])
    drain_rows(sS[(CHUNKS_PT - 1) % 3])

  def count_pass():
    def fire_i(k, t):
      r0 = irow0 + k * SUBS_PER_CHUNK
      return pltpu.async_copy(sidx.at[pl.ds(r0, SUBS_PER_CHUNK)],
                              sis[t], sI[t])

    def drain_i(t):
      pltpu.make_async_copy(sidx.at[pl.ds(irow0, SUBS_PER_CHUNK)],
                            sis[0], sI[t]).wait()

    def fire_s(t):
      for j in range(SUBS_PER_CHUNK):
        pltpu.async_copy(obuf, acc.at[sis[t].at[j]], sS[t], add=True)

    def drain_s(t):
      pltpu.make_async_copy(feat_all.at[pl.ds(0, CHUNK)], rows[0],
                            sS[t]).wait()

    fire_i(0, 0).wait()
    def body(m, _):
      for t in range(3):
        k = 3 * m + t
        nxt = (t + 1) % 3
        if t < 2:
          @pl.when(m > 0)
          def _():
            drain_s(nxt)
          fire_i(k + 1, nxt)
        else:
          drain_s(nxt)
          @pl.when(m < TRIPLES - 1)
          def _():
            fire_i(k + 1, nxt)
        @pl.when(k > 0)
        def _():
          drain_i(t)
        fire_s(t)
      return 0
    lax.fori_loop(0, TRIPLES, body, 0)
    drain_s((CHUNKS_PT - 2) % 3)
    drain_s((CHUNKS_PT - 1) % 3)

  def make_divide(buf):
    def divide(r):
      def divgrp(k, _):
        rvec = recip_v[pl.ds(r * NODES_PT_PAD + k * 16, 16)]
        base = k * 16
        for j in range(16):
          rv = rvec[j]
          buf[base + j, 0:16] = buf[base + j, 0:16] * rv
        return 0
      lax.fori_loop(0, NODES_PT_PAD // 16, divgrp, 0)
    return divide

  divides = [make_divide(rows[0]), make_divide(rows[1]),
             make_divide(rows[2])]

  def fire_zero_rel(r):
    row0 = r * N_NODES + nbase
    ds = [pltpu.async_copy(zbuf, acc.at[pl.ds(row0, ZBUF_ROWS)], sS[0]),
          pltpu.async_copy(zbuf,
                           acc.at[pl.ds(row0 + ZBUF_ROWS, ZBUF_ROWS)], sS[0]),
          pltpu.async_copy(zbuf.at[pl.ds(0, NODES_PT - 2 * ZBUF_ROWS)],
                           acc.at[pl.ds(row0 + 2 * ZBUF_ROWS,
                                        NODES_PT - 2 * ZBUF_ROWS)], sS[0])]
    return ds

  def fire_zero_dump():
    return [pltpu.async_copy(
        zbuf.at[pl.ds(0, PAD_ROWS // NTILES)],
        acc.at[pl.ds(ROWS + s * (PAD_ROWS // NTILES), PAD_ROWS // NTILES)],
        sS[0])]

  def fire_in(r):
    row0 = r * N_NODES + nbase
    return pltpu.async_copy(acc.at[pl.ds(row0, NODES_PT)],
                            rows[r % 3].at[pl.ds(0, NODES_PT)], sI[r % 3])

  def writeback(dst_for):
    zs = fire_zero_dump()
    ins = {0: fire_in(0)}
    outs = {}
    for r in range(NUM_REL):
      buf = rows[r % 3]
      if r + 1 < NUM_REL:
        if r - 2 >= 0:
          outs[r - 2].wait()
        ins[r + 1] = fire_in(r + 1)
      ins[r].wait()
      zs += fire_zero_rel(r)
      divides[r % 3](r)
      outs[r] = pltpu.async_copy(buf.at[pl.ds(0, NODES_PT)],
                                 dst_for(r), sG[r % 3])
    for r in (NUM_REL - 3, NUM_REL - 2, NUM_REL - 1):
      outs[r].wait()
    for d in zs:
      d.wait()

  def zfill(i, _):
    zbuf[i, 0:16] = zvec
    return 0
  lax.fori_loop(0, ZBUF_ROWS, zfill, 0)
  def ones_fill(i, _):
    obuf[i, 0:16] = onevec
    return 0
  lax.fori_loop(0, SUB, ones_fill, 0)
  zero_acc()
  plsc.subcore_barrier()

  count_pass()
  plsc.subcore_barrier()
  zcol = jnp.zeros((16,), jnp.int32)
  zs0 = fire_zero_dump()
  for r in range(NUM_REL):
    row0 = r * N_NODES + nbase
    pltpu.sync_copy(acc.at[pl.ds(row0, NODES_PT)],
                    rows[0].at[pl.ds(0, NODES_PT)])
    zs0 += fire_zero_rel(r)
    def recipgrp(k, _, r=r):
      ridx = k * 16 + lane
      cv = plsc.load_gather(rows[0], [ridx, zcol])
      rvec = 1.0 / jnp.maximum(cv, 1.0)
      recip_v[pl.ds(r * NODES_PT_PAD + k * 16, 16)] = rvec
      return 0
    lax.fori_loop(0, NODES_PT_PAD // 16, recipgrp, 0)
  for d in zs0:
    d.wait()
  plsc.subcore_barrier()

  def step1(j, _):
    p = c * PASS_PC + j
    scatter_pass(feat_all, gidx1, p * N_NODES)
    plsc.subcore_barrier()
    writeback(lambda r: h1_all.at[
        pl.ds(p * ROWS + r * N_NODES + nbase, NODES_PT)])
    plsc.subcore_barrier()
    return 0
  lax.fori_loop(0, PASS_PC, step1, 0)

  def step2(j, _):
    p = c * PASS_PC + j
    scatter_pass(h1_all, gidx2, p * ROWS)
    plsc.subcore_barrier()
    writeback(lambda r: h2.at[pl.ds(nbase, NODES_PT),
                              pl.ds(r * D + p * COL, COL)])
    plsc.subcore_barrier()
    return 0
  lax.fori_loop(0, PASS_PC, step2, 0)


def _matmul_body(h2_ref, w_ref, o_ref):
  o_ref[...] = jax.nn.relu(
      jnp.dot(h2_ref[...], w_ref[...], preferred_element_type=jnp.float32))


def kernel(features, edge_index, edge_type, W):
  n = features.shape[0]
  loop = jnp.arange(n, dtype=edge_index.dtype)
  src = jnp.concatenate([edge_index[0], loop])
  dst = jnp.concatenate([edge_index[1], loop])
  etype = jnp.concatenate([edge_type, jnp.zeros((n,), edge_type.dtype)])

  sidx = etype * n + dst
  g2 = etype * n + src

  pad = E_PAD - E_TOT
  padi = jnp.arange(pad, dtype=jnp.int32)
  src_p = jnp.concatenate([src, padi % n]).reshape(E_PAD // SUB, SUB)
  sidx_p = jnp.concatenate([sidx, ROWS + padi % PAD_ROWS]).reshape(
      E_PAD // SUB, SUB)
  g2_p = jnp.concatenate([g2, padi % ROWS]).reshape(E_PAD // SUB, SUB)

  feat_all = jnp.transpose(
      features.reshape(n, NPASS, COL), (1, 0, 2)).reshape(NPASS * n, COL)

  mesh = plsc.VectorSubcoreMesh(core_axis_name="c", subcore_axis_name="s")
  f32 = jnp.float32
  sc = pl.kernel(
      _sc_body,
      out_type=(
          jax.ShapeDtypeStruct((n, NUM_REL * D), f32),
          jax.ShapeDtypeStruct((NPASS * ROWS, COL), f32),
      ),
      mesh=mesh,
      scratch_types=[
          pltpu.VMEM_SHARED((ACC_ROWS, COL), f32),
          pltpu.VMEM((ZBUF_ROWS, COL), f32),
          pltpu.VMEM((SUB, COL), f32),
          [pltpu.VMEM((CHUNK, COL), f32) for _ in range(3)],
          [pltpu.VMEM((SUBS_PER_CHUNK, SUB), jnp.int32)
           for _ in range(3)],
          [pltpu.VMEM((SUBS_PER_CHUNK, SUB), jnp.int32)
           for _ in range(3)],
          pltpu.VMEM((NUM_REL * NODES_PT_PAD,), f32),
          pltpu.SemaphoreType.DMA,
          [pltpu.SemaphoreType.DMA for _ in range(3)],
          [pltpu.SemaphoreType.DMA for _ in range(3)],
          [pltpu.SemaphoreType.DMA for _ in range(3)],
      ],
      compiler_params=pltpu.CompilerParams(use_tc_tiling_on_sc=False,
                                           needs_layout_passes=False),
      name="rhdc_sc_diffusion",
  )
  h2, _ = sc(feat_all, src_p, sidx_p, g2_p)

  w_flat = W.reshape(NUM_REL * D, D)
  block_m = 400
  out = pl.pallas_call(
      _matmul_body,
      grid=(n // block_m,),
      in_specs=[
          pl.BlockSpec((block_m, NUM_REL * D), lambda i: (i, 0)),
          pl.BlockSpec((NUM_REL * D, D), lambda i: (0, 0)),
      ],
      out_specs=pl.BlockSpec((block_m, D), lambda i: (i, 0)),
      out_shape=jax.ShapeDtypeStruct((n, D), jnp.float32),
  )(h2, w_flat)
  return out
